# baseline (speedup 1.0000x reference)
import jax
import jax.numpy as jnp
from jax import lax
from jax.experimental import pallas as pl
from jax.experimental.pallas import tpu as pltpu
from jax.experimental.pallas import tpu_sc as plsc

_S = 8192
_C = 1000
_D = 4096
_NW = 32
_W = _D // _NW
_K = 16
_CHUNKS = _S // _K


def _sc_segsum_body(flat3, labels2, sums3, acc, idx_v, buf):
    cid = lax.axis_index("c")
    sid = lax.axis_index("s")
    wid = sid * 2 + cid
    zero16 = jnp.zeros((16,), jnp.float32)

    def _zc(r, carry):
        for c in range(_W // 16):
            acc[r, pl.ds(c * 16, 16)] = zero16
        return carry

    lax.fori_loop(0, _C, _zc, 0)

    def _chunk(j, carry):
        pltpu.sync_copy(labels2.at[j], idx_v)
        pltpu.sync_copy(flat3.at[pl.ds(j * _K, _K), wid], buf)
        lbl = idx_v[...]
        for r in range(_K):
            row = lbl[r]
            for c in range(_W // 16):
                plsc.addupdate(acc.at[row, pl.ds(c * 16, 16)],
                               buf[r, pl.ds(c * 16, 16)])
        return carry

    lax.fori_loop(0, _CHUNKS, _chunk, 0)

    pltpu.sync_copy(acc, sums3.at[:, wid])


_sc_segsum = pl.kernel(
    _sc_segsum_body,
    out_type=jax.ShapeDtypeStruct((_C, _NW, _W), jnp.float32),
    mesh=plsc.VectorSubcoreMesh(core_axis_name="c", subcore_axis_name="s"),
    scratch_types=[
        pltpu.VMEM((_C, _W), jnp.float32),
        pltpu.VMEM((_K,), jnp.int32),
        pltpu.VMEM((_K, _W), jnp.float32),
    ],
)


def _ema_body(sums_ref, labels_ref, protos_ref, pc_ref, out_ref):
    blk = out_ref.shape[0]
    i = pl.program_id(0)
    lbl = labels_ref[...]
    cls = i * blk + jax.lax.broadcasted_iota(jnp.int32, (blk, 1, 1), 0)
    onehot = (lbl[None, :, :] == cls).astype(jnp.float32)
    cnt = jnp.sum(onehot, axis=(1, 2))[:, None]
    a = 1.0 / (pc_ref[:, 0:1] + 1.0)
    mean = sums_ref[...] / jnp.maximum(cnt, 1.0)
    upd = (1.0 - a) * protos_ref[...] + a * mean
    out_ref[...] = jnp.where(cnt > 0.0, upd, protos_ref[...])


def kernel(x, support_examples, support_labels, num_shots, class_prototypes, prototype_counts):
    flat3 = support_examples.reshape(_S, _NW, _W)
    labels2 = support_labels.reshape(_CHUNKS, _K)

    sums = _sc_segsum(flat3, labels2).reshape(_C, _D)

    blk = 200
    labels2d = support_labels.reshape(16, 512)
    pc_b = jnp.broadcast_to(prototype_counts[:, None], (_C, 16))
    new_protos = pl.pallas_call(
        _ema_body,
        grid=(_C // blk,),
        in_specs=[
            pl.BlockSpec((blk, _D), lambda i: (i, 0)),
            pl.BlockSpec((16, 512), lambda i: (0, 0)),
            pl.BlockSpec((blk, _D), lambda i: (i, 0)),
            pl.BlockSpec((blk, 16), lambda i: (i, 0)),
        ],
        out_specs=pl.BlockSpec((blk, _D), lambda i: (i, 0)),
        out_shape=jax.ShapeDtypeStruct((_C, _D), jnp.float32),
    )(sums, labels2d, class_prototypes, pc_b)

    return x, new_protos


# speedup vs baseline: 1.1911x; 1.1911x over previous
import jax
import jax.numpy as jnp
from jax import lax
from jax.experimental import pallas as pl
from jax.experimental.pallas import tpu as pltpu
from jax.experimental.pallas import tpu_sc as plsc

_S = 8192
_C = 1000
_D = 4096
_NSL = 64
_W = _D // _NSL
_K = 128
_NCH = _S // _K


def _acc_chunk(acc, lab_v, buf, chunk):
    for g in range(_K // 16):
        lbl = lab_v[pl.ds(chunk * _K + g * 16, 16)]
        for r in range(16):
            row = lbl[r]
            for c in range(_W // 16):
                plsc.addupdate(acc.at[pl.ds(row * _W + c * 16, 16)],
                               buf[g * 16 + r, pl.ds(c * 16, 16)])


def _sc_segsum_body(flat3, labels1, sums2, acc, lab_v, buf0, buf1, sem0, sem1):
    cid = lax.axis_index("c")
    sid = lax.axis_index("s")
    wid = sid * 2 + cid
    zero16 = jnp.zeros((16,), jnp.float32)

    pltpu.sync_copy(labels1, lab_v)

    for p in range(2):
        s = p * 32 + wid

        def _zc(i, carry):
            for c in range(_W // 16):
                acc[pl.ds(i * _W + c * 16, 16)] = zero16
            return carry

        lax.fori_loop(0, _C, _zc, 0)

        pltpu.async_copy(flat3.at[pl.ds(0, _K), s], buf0, sem0)

        def _two(i, carry):
            j = 2 * i
            pltpu.make_async_copy(flat3.at[pl.ds(j * _K, _K), s], buf0, sem0).wait()
            pltpu.async_copy(flat3.at[pl.ds((j + 1) * _K, _K), s], buf1, sem1)
            _acc_chunk(acc, lab_v, buf0, j)
            pltpu.make_async_copy(flat3.at[pl.ds((j + 1) * _K, _K), s], buf1, sem1).wait()

            @pl.when(i < _NCH // 2 - 1)
            def _():
                pltpu.async_copy(flat3.at[pl.ds((j + 2) * _K, _K), s], buf0, sem0)

            _acc_chunk(acc, lab_v, buf1, j + 1)
            return carry

        lax.fori_loop(0, _NCH // 2, _two, 0)

        pltpu.sync_copy(acc, sums2.at[s])


_sc_segsum = pl.kernel(
    _sc_segsum_body,
    out_type=jax.ShapeDtypeStruct((_NSL, _C * _W), jnp.float32),
    mesh=plsc.VectorSubcoreMesh(core_axis_name="c", subcore_axis_name="s"),
    scratch_types=[
        pltpu.VMEM((_C * _W,), jnp.float32),
        pltpu.VMEM((_S,), jnp.int32),
        pltpu.VMEM((_K, _W), jnp.float32),
        pltpu.VMEM((_K, _W), jnp.float32),
        pltpu.SemaphoreType.DMA,
        pltpu.SemaphoreType.DMA,
    ],
)


def _ema_body(sums_ref, labels_ref, protos_ref, pc_ref, out_ref):
    blk = out_ref.shape[0]
    i = pl.program_id(0)
    lbl = labels_ref[...]
    cls = i * blk + jax.lax.broadcasted_iota(jnp.int32, (blk, 1, 1), 0)
    onehot = (lbl[None, :, :] == cls).astype(jnp.float32)
    cnt = jnp.sum(onehot, axis=(1, 2))[:, None]
    a = 1.0 / (pc_ref[:, 0:1] + 1.0)
    mean = sums_ref[...] / jnp.maximum(cnt, 1.0)
    upd = (1.0 - a) * protos_ref[...] + a * mean
    out_ref[...] = jnp.where(cnt > 0.0, upd, protos_ref[...])


def kernel(x, support_examples, support_labels, num_shots, class_prototypes, prototype_counts):
    flat3 = support_examples.reshape(_S, _NSL, _W)

    sums2 = _sc_segsum(flat3, support_labels)
    sums = sums2.reshape(_NSL, _C, _W).transpose(1, 0, 2).reshape(_C, _D)

    blk = 200
    labels2d = support_labels.reshape(16, 512)
    pc_b = jnp.broadcast_to(prototype_counts[:, None], (_C, 16))
    new_protos = pl.pallas_call(
        _ema_body,
        grid=(_C // blk,),
        in_specs=[
            pl.BlockSpec((blk, _D), lambda i: (i, 0)),
            pl.BlockSpec((16, 512), lambda i: (0, 0)),
            pl.BlockSpec((blk, _D), lambda i: (i, 0)),
            pl.BlockSpec((blk, 16), lambda i: (i, 0)),
        ],
        out_specs=pl.BlockSpec((blk, _D), lambda i: (i, 0)),
        out_shape=jax.ShapeDtypeStruct((_C, _D), jnp.float32),
    )(sums, labels2d, class_prototypes, pc_b)

    return x, new_protos


# speedup vs baseline: 1.4369x; 1.2063x over previous
import jax
import jax.numpy as jnp
from jax import lax
from jax.experimental import pallas as pl
from jax.experimental.pallas import tpu as pltpu
from jax.experimental.pallas import tpu_sc as plsc

_S = 8192
_C = 1000
_D = 4096
_NSL = 64
_W = _D // _NSL
_K = 128
_NCH = _S // _K


def _acc_chunk(acc, lab_v, buf, chunk):
    @plsc.parallel_loop(0, _K // 16, unroll=2)
    def _grp(g):
        lbl = lab_v[pl.ds(chunk * _K + g * 16, 16)]
        for r in range(16):
            row = lbl[r]
            for c in range(_W // 16):
                plsc.addupdate(acc.at[pl.ds(row * _W + c * 16, 16)],
                               buf[g * 16 + r, pl.ds(c * 16, 16)])


def _sc_segsum_body(flat3, labels1, sums2, acc, lab_v, buf0, buf1, sem0, sem1):
    cid = lax.axis_index("c")
    sid = lax.axis_index("s")
    wid = sid * 2 + cid
    zero16 = jnp.zeros((16,), jnp.float32)

    pltpu.sync_copy(labels1, lab_v)

    for p in range(2):
        s = p * 32 + wid

        def _zc(i, carry):
            for c in range(_W // 16):
                acc[pl.ds(i * _W + c * 16, 16)] = zero16
            return carry

        lax.fori_loop(0, _C, _zc, 0)

        pltpu.async_copy(flat3.at[pl.ds(0, _K), s], buf0, sem0)

        def _two(i, carry):
            j = 2 * i
            pltpu.make_async_copy(flat3.at[pl.ds(j * _K, _K), s], buf0, sem0).wait()
            pltpu.async_copy(flat3.at[pl.ds((j + 1) * _K, _K), s], buf1, sem1)
            _acc_chunk(acc, lab_v, buf0, j)
            pltpu.make_async_copy(flat3.at[pl.ds((j + 1) * _K, _K), s], buf1, sem1).wait()

            @pl.when(i < _NCH // 2 - 1)
            def _():
                pltpu.async_copy(flat3.at[pl.ds((j + 2) * _K, _K), s], buf0, sem0)

            _acc_chunk(acc, lab_v, buf1, j + 1)
            return carry

        lax.fori_loop(0, _NCH // 2, _two, 0)

        pltpu.sync_copy(acc, sums2.at[s])


_sc_segsum = pl.kernel(
    _sc_segsum_body,
    out_type=jax.ShapeDtypeStruct((_NSL, _C * _W), jnp.float32),
    mesh=plsc.VectorSubcoreMesh(core_axis_name="c", subcore_axis_name="s"),
    scratch_types=[
        pltpu.VMEM((_C * _W,), jnp.float32),
        pltpu.VMEM((_S,), jnp.int32),
        pltpu.VMEM((_K, _W), jnp.float32),
        pltpu.VMEM((_K, _W), jnp.float32),
        pltpu.SemaphoreType.DMA,
        pltpu.SemaphoreType.DMA,
    ],
)


def _ema_body(sums_ref, labels_ref, protos_ref, pc_ref, out_ref):
    blk = out_ref.shape[0]
    i = pl.program_id(0)
    lbl = labels_ref[...]
    cls = i * blk + jax.lax.broadcasted_iota(jnp.int32, (blk, 1, 1), 0)
    onehot = (lbl[None, :, :] == cls).astype(jnp.float32)
    cnt = jnp.sum(onehot, axis=(1, 2))[:, None]
    a = 1.0 / (pc_ref[:, 0:1] + 1.0)
    mean = sums_ref[...] / jnp.maximum(cnt, 1.0)
    upd = (1.0 - a) * protos_ref[...] + a * mean
    out_ref[...] = jnp.where(cnt > 0.0, upd, protos_ref[...])


def kernel(x, support_examples, support_labels, num_shots, class_prototypes, prototype_counts):
    flat3 = support_examples.reshape(_S, _NSL, _W)

    sums2 = _sc_segsum(flat3, support_labels)
    sums = sums2.reshape(_NSL, _C, _W).transpose(1, 0, 2).reshape(_C, _D)

    blk = 200
    labels2d = support_labels.reshape(16, 512)
    pc_b = jnp.broadcast_to(prototype_counts[:, None], (_C, 16))
    new_protos = pl.pallas_call(
        _ema_body,
        grid=(_C // blk,),
        in_specs=[
            pl.BlockSpec((blk, _D), lambda i: (i, 0)),
            pl.BlockSpec((16, 512), lambda i: (0, 0)),
            pl.BlockSpec((blk, _D), lambda i: (i, 0)),
            pl.BlockSpec((blk, 16), lambda i: (i, 0)),
        ],
        out_specs=pl.BlockSpec((blk, _D), lambda i: (i, 0)),
        out_shape=jax.ShapeDtypeStruct((_C, _D), jnp.float32),
    )(sums, labels2d, class_prototypes, pc_b)

    return x, new_protos


# speedup vs baseline: 1.4797x; 1.0298x over previous
import jax
import jax.numpy as jnp
from jax import lax
from jax.experimental import pallas as pl
from jax.experimental.pallas import tpu as pltpu
from jax.experimental.pallas import tpu_sc as plsc

_S = 8192
_C = 1000
_D = 4096

_SC_S = 2048
_TC_S = _S - _SC_S
_R0 = _TC_S

_NSL = 64
_W = _D // _NSL
_K = 128
_NCH = _SC_S // _K

_C_PAD = 1024
_S_BLK = 512
_F_BLK = 2048


def _acc_chunk(acc, lab_v, buf, chunk):
    @plsc.parallel_loop(0, _K // 16, unroll=2)
    def _grp(g):
        lbl = lab_v[pl.ds(_R0 + chunk * _K + g * 16, 16)]
        for r in range(16):
            row = lbl[r]
            for c in range(_W // 16):
                plsc.addupdate(acc.at[pl.ds(row * _W + c * 16, 16)],
                               buf[g * 16 + r, pl.ds(c * 16, 16)])


def _sc_segsum_body(flat3, labels1, sums2, acc, lab_v, buf0, buf1, sem0, sem1):
    cid = lax.axis_index("c")
    sid = lax.axis_index("s")
    wid = sid * 2 + cid
    zero16 = jnp.zeros((16,), jnp.float32)

    pltpu.sync_copy(labels1, lab_v)

    for p in range(2):
        s = p * 32 + wid

        def _zc(i, carry):
            for c in range(_W // 16):
                acc[pl.ds(i * _W + c * 16, 16)] = zero16
            return carry

        lax.fori_loop(0, _C, _zc, 0)

        pltpu.async_copy(flat3.at[pl.ds(_R0, _K), s], buf0, sem0)

        def _two(i, carry):
            j = 2 * i
            pltpu.make_async_copy(flat3.at[pl.ds(_R0 + j * _K, _K), s], buf0, sem0).wait()
            pltpu.async_copy(flat3.at[pl.ds(_R0 + (j + 1) * _K, _K), s], buf1, sem1)
            _acc_chunk(acc, lab_v, buf0, j)
            pltpu.make_async_copy(flat3.at[pl.ds(_R0 + (j + 1) * _K, _K), s], buf1, sem1).wait()

            @pl.when(i < _NCH // 2 - 1)
            def _():
                pltpu.async_copy(flat3.at[pl.ds(_R0 + (j + 2) * _K, _K), s], buf0, sem0)

            _acc_chunk(acc, lab_v, buf1, j + 1)
            return carry

        lax.fori_loop(0, _NCH // 2, _two, 0)

        pltpu.sync_copy(acc, sums2.at[s])


_sc_segsum = pl.kernel(
    _sc_segsum_body,
    out_type=jax.ShapeDtypeStruct((_NSL, _C * _W), jnp.float32),
    mesh=plsc.VectorSubcoreMesh(core_axis_name="c", subcore_axis_name="s"),
    scratch_types=[
        pltpu.VMEM((_C * _W,), jnp.float32),
        pltpu.VMEM((_S,), jnp.int32),
        pltpu.VMEM((_K, _W), jnp.float32),
        pltpu.VMEM((_K, _W), jnp.float32),
        pltpu.SemaphoreType.DMA,
        pltpu.SemaphoreType.DMA,
    ],
)


def _tc_segsum_body(labels_ref, flat_ref, sums_ref):
    f = pl.program_id(0)
    s = pl.program_id(1)
    lbl = labels_ref[0, 0, :]
    cls = jax.lax.broadcasted_iota(jnp.int32, (_C_PAD, _S_BLK), 0)
    onehot = (lbl[None, :] == cls).astype(jnp.float32)
    partial = jnp.dot(onehot, flat_ref[...], preferred_element_type=jnp.float32)

    @pl.when(s == 0)
    def _():
        sums_ref[...] = partial

    @pl.when(s != 0)
    def _():
        sums_ref[...] += partial


def _ema_body(sums_tc_ref, sums_sc_ref, labels_ref, protos_ref, pc_ref, out_ref):
    blk = out_ref.shape[0]
    i = pl.program_id(0)
    lbl = labels_ref[...]
    cls = i * blk + jax.lax.broadcasted_iota(jnp.int32, (blk, 1, 1), 0)
    onehot = (lbl[None, :, :] == cls).astype(jnp.float32)
    cnt = jnp.sum(onehot, axis=(1, 2))[:, None]
    a = 1.0 / (pc_ref[:, 0:1] + 1.0)
    sums = sums_tc_ref[...] + sums_sc_ref[...]
    mean = sums / jnp.maximum(cnt, 1.0)
    upd = (1.0 - a) * protos_ref[...] + a * mean
    out_ref[...] = jnp.where(cnt > 0.0, upd, protos_ref[...])


def kernel(x, support_examples, support_labels, num_shots, class_prototypes, prototype_counts):
    flat3 = support_examples.reshape(_S, _NSL, _W)

    sums2 = _sc_segsum(flat3, support_labels)
    sums_sc = sums2.reshape(_NSL, _C, _W).transpose(1, 0, 2).reshape(_C, _D)

    flat_tc = support_examples.reshape(_S, _D)[:_TC_S]
    labels_tc = support_labels[:_TC_S].reshape(_TC_S // _S_BLK, 1, _S_BLK)
    sums_tc = pl.pallas_call(
        _tc_segsum_body,
        grid=(_D // _F_BLK, _TC_S // _S_BLK),
        in_specs=[
            pl.BlockSpec((1, 1, _S_BLK), lambda f, s: (s, 0, 0)),
            pl.BlockSpec((_S_BLK, _F_BLK), lambda f, s: (s, f)),
        ],
        out_specs=pl.BlockSpec((_C_PAD, _F_BLK), lambda f, s: (0, f)),
        out_shape=jax.ShapeDtypeStruct((_C_PAD, _D), jnp.float32),
    )(labels_tc, flat_tc)

    blk = 200
    labels2d = support_labels.reshape(16, 512)
    pc_b = jnp.broadcast_to(prototype_counts[:, None], (_C, 16))
    new_protos = pl.pallas_call(
        _ema_body,
        grid=(_C // blk,),
        in_specs=[
            pl.BlockSpec((blk, _D), lambda i: (i, 0)),
            pl.BlockSpec((blk, _D), lambda i: (i, 0)),
            pl.BlockSpec((16, 512), lambda i: (0, 0)),
            pl.BlockSpec((blk, _D), lambda i: (i, 0)),
            pl.BlockSpec((blk, 16), lambda i: (i, 0)),
        ],
        out_specs=pl.BlockSpec((blk, _D), lambda i: (i, 0)),
        out_shape=jax.ShapeDtypeStruct((_C, _D), jnp.float32),
    )(sums_tc, sums_sc, labels2d, class_prototypes, pc_b)

    return x, new_protos


# speedup vs baseline: 1.8310x; 1.2374x over previous
import jax
import jax.numpy as jnp
from jax import lax
from jax.experimental import pallas as pl
from jax.experimental.pallas import tpu as pltpu
from jax.experimental.pallas import tpu_sc as plsc

_S = 8192
_C = 1000
_D = 4096

_SC_S = 2048
_TC_S = _S - _SC_S
_R0 = _TC_S

_NSL = 64
_W = _D // _NSL
_K = 128
_NCH = _SC_S // _K

_C_PAD = 1024
_S_BLK = 512
_F_BLK = 2048


def _acc_chunk(acc, lab_v, buf, chunk):
    @plsc.parallel_loop(0, _K // 16, unroll=2)
    def _grp(g):
        lbl = lab_v[pl.ds(_R0 + chunk * _K + g * 16, 16)]
        for r in range(16):
            row = lbl[r]
            for c in range(_W // 16):
                plsc.addupdate(acc.at[pl.ds(row * _W + c * 16, 16)],
                               buf[g * 16 + r, pl.ds(c * 16, 16)])


def _sc_segsum_body(flat3, labels1, sums2, acc, lab_v, buf0, buf1, sem0, sem1):
    cid = lax.axis_index("c")
    sid = lax.axis_index("s")
    wid = sid * 2 + cid
    zero16 = jnp.zeros((16,), jnp.float32)

    pltpu.sync_copy(labels1, lab_v)

    for p in range(2):
        s = p * 32 + wid

        def _zc(i, carry):
            for c in range(_W // 16):
                acc[pl.ds(i * _W + c * 16, 16)] = zero16
            return carry

        lax.fori_loop(0, _C, _zc, 0)

        pltpu.async_copy(flat3.at[pl.ds(_R0, _K), s], buf0, sem0)

        def _two(i, carry):
            j = 2 * i
            pltpu.make_async_copy(flat3.at[pl.ds(_R0 + j * _K, _K), s], buf0, sem0).wait()
            pltpu.async_copy(flat3.at[pl.ds(_R0 + (j + 1) * _K, _K), s], buf1, sem1)
            _acc_chunk(acc, lab_v, buf0, j)
            pltpu.make_async_copy(flat3.at[pl.ds(_R0 + (j + 1) * _K, _K), s], buf1, sem1).wait()

            @pl.when(i < _NCH // 2 - 1)
            def _():
                pltpu.async_copy(flat3.at[pl.ds(_R0 + (j + 2) * _K, _K), s], buf0, sem0)

            _acc_chunk(acc, lab_v, buf1, j + 1)
            return carry

        lax.fori_loop(0, _NCH // 2, _two, 0)

        pltpu.sync_copy(acc, sums2.at[s])


_sc_segsum = pl.kernel(
    _sc_segsum_body,
    out_type=jax.ShapeDtypeStruct((_NSL, _C * _W), jnp.float32),
    mesh=plsc.VectorSubcoreMesh(core_axis_name="c", subcore_axis_name="s"),
    scratch_types=[
        pltpu.VMEM((_C * _W,), jnp.float32),
        pltpu.VMEM((_S,), jnp.int32),
        pltpu.VMEM((_K, _W), jnp.float32),
        pltpu.VMEM((_K, _W), jnp.float32),
        pltpu.SemaphoreType.DMA,
        pltpu.SemaphoreType.DMA,
    ],
)


def _tc_segsum_body(labels_ref, flat_ref, sums_ref):
    f = pl.program_id(0)
    s = pl.program_id(1)
    lbl = labels_ref[0, 0, :]
    cls = jax.lax.broadcasted_iota(jnp.int32, (_C_PAD, _S_BLK), 0)
    onehot = (lbl[None, :] == cls).astype(jnp.float32)
    partial = jnp.dot(onehot, flat_ref[...], preferred_element_type=jnp.float32)

    @pl.when(s == 0)
    def _():
        sums_ref[...] = partial

    @pl.when(s != 0)
    def _():
        sums_ref[...] += partial


def _ema_body(sums_tc_ref, sums_sc_ref, labels_ref, protos_ref, pc_ref, out_ref):
    blk = out_ref.shape[0]
    i = pl.program_id(0)
    lbl = labels_ref[...]
    cls = i * blk + jax.lax.broadcasted_iota(jnp.int32, (blk, 1, 1), 0)
    onehot = (lbl[None, :, :] == cls).astype(jnp.float32)
    cnt = jnp.sum(onehot, axis=(1, 2))[:, None]
    a = 1.0 / (pc_ref[:, 0:1] + 1.0)
    ssc = jnp.swapaxes(sums_sc_ref[...], 0, 1).reshape(blk, _D)
    sums = sums_tc_ref[...] + ssc
    mean = sums / jnp.maximum(cnt, 1.0)
    upd = (1.0 - a) * protos_ref[...] + a * mean
    out_ref[...] = jnp.where(cnt > 0.0, upd, protos_ref[...])


def kernel(x, support_examples, support_labels, num_shots, class_prototypes, prototype_counts):
    flat3 = support_examples.reshape(_S, _NSL, _W)

    sums2 = _sc_segsum(flat3, support_labels)
    sums_sc3 = sums2.reshape(_NSL, _C, _W)

    flat_tc = support_examples.reshape(_S, _D)
    labels_tc = support_labels.reshape(_S // _S_BLK, 1, _S_BLK)
    sums_tc = pl.pallas_call(
        _tc_segsum_body,
        grid=(_D // _F_BLK, _TC_S // _S_BLK),
        in_specs=[
            pl.BlockSpec((1, 1, _S_BLK), lambda f, s: (s, 0, 0)),
            pl.BlockSpec((_S_BLK, _F_BLK), lambda f, s: (s, f)),
        ],
        out_specs=pl.BlockSpec((_C_PAD, _F_BLK), lambda f, s: (0, f)),
        out_shape=jax.ShapeDtypeStruct((_C_PAD, _D), jnp.float32),
    )(labels_tc, flat_tc)

    blk = 200
    labels2d = support_labels.reshape(16, 512)
    pc_b = jnp.broadcast_to(prototype_counts[:, None], (_C, 16))
    new_protos = pl.pallas_call(
        _ema_body,
        grid=(_C // blk,),
        in_specs=[
            pl.BlockSpec((blk, _D), lambda i: (i, 0)),
            pl.BlockSpec((_NSL, blk, _W), lambda i: (0, i, 0)),
            pl.BlockSpec((16, 512), lambda i: (0, 0)),
            pl.BlockSpec((blk, _D), lambda i: (i, 0)),
            pl.BlockSpec((blk, 16), lambda i: (i, 0)),
        ],
        out_specs=pl.BlockSpec((blk, _D), lambda i: (i, 0)),
        out_shape=jax.ShapeDtypeStruct((_C, _D), jnp.float32),
    )(sums_tc, sums_sc3, labels2d, class_prototypes, pc_b)

    return x, new_protos
